# baseline (speedup 1.0000x reference)
import functools

import jax
import jax.numpy as jnp
from jax import lax
from jax.experimental import pallas as pl
from jax.experimental.pallas import tpu as pltpu
from jax.experimental.pallas import tpu_sc as plsc

P = 1000
E = 512
F = 20
SEQ = 2048
TOT = P + SEQ

_NC = 2
_NS = 16
_NW = _NC * _NS

_GATHER_WORKERS = 25
_GATHER_ROWS = P // _GATHER_WORKERS
_EMB_ROWS = SEQ // _NW


def _y_body(table_hbm, labels_hbm, emby_hbm, out_hbm, idx_v, rows_v, buf_v,
            gsem):
    wid = lax.axis_index("s") * _NC + lax.axis_index("c")

    @pl.when(wid < _GATHER_WORKERS)
    def _():
        base = wid * _GATHER_ROWS
        pltpu.sync_copy(labels_hbm.at[0], idx_v)
        pltpu.async_copy(
            table_hbm.at[idx_v.at[pl.ds(base, _GATHER_ROWS)]], rows_v,
            gsem).wait()
        pltpu.sync_copy(rows_v, out_hbm.at[0, pl.ds(base, _GATHER_ROWS)])

    ebase = wid * _EMB_ROWS
    pltpu.sync_copy(emby_hbm.at[0, pl.ds(ebase, _EMB_ROWS)], buf_v)
    pltpu.sync_copy(buf_v, out_hbm.at[0, pl.ds(P + ebase, _EMB_ROWS)])


@functools.cache
def _y_concat():
    return pl.kernel(
        _y_body,
        out_type=jax.ShapeDtypeStruct((1, TOT, E), jnp.float32),
        mesh=plsc.VectorSubcoreMesh(core_axis_name="c", subcore_axis_name="s"),
        scratch_types=[
            pltpu.VMEM((P,), jnp.int32),
            pltpu.VMEM((_GATHER_ROWS, E), jnp.float32),
            pltpu.VMEM((_EMB_ROWS, E), jnp.float32),
            pltpu.SemaphoreType.DMA,
        ],
    )

_HR = SEQ // 2
_NTC = F
_NCH = F + 2 * F
_NBUF = 24
_DEPTH = 12


def _x_pieces():
    pieces = []
    for f in range(F):
        pieces.append(("tune", (0, f, 0), (0, f, 0), P))
    for f in range(F):
        for h in range(2):
            pieces.append(("emb", (0, f, h * _HR), (0, f, P + h * _HR), _HR))
    return pieces


def _x_copy(k, tune_ref, emb_ref, out_ref, buf):
    kind, (s0, s1, s2), (d0, d1, d2), rows = _x_pieces()[k]
    src_ref = tune_ref if kind == "tune" else emb_ref
    b = k % _NBUF
    src = src_ref.at[s0, s1, pl.ds(s2, rows)]
    dst = out_ref.at[d0, d1, pl.ds(d2, rows)]
    stage = buf.at[b, pl.ds(0, rows)]
    return src, stage, dst


def _x_body(tune_ref, emb_ref, out_ref, buf, in_sems, out_sems):
    def in_copy(k):
        src, stage, _ = _x_copy(k, tune_ref, emb_ref, out_ref, buf)
        return pltpu.make_async_copy(src, stage, in_sems.at[k % _NBUF])

    def out_copy(k):
        _, stage, dst = _x_copy(k, tune_ref, emb_ref, out_ref, buf)
        return pltpu.make_async_copy(stage, dst, out_sems.at[k % _NBUF])

    for k in range(_DEPTH):
        in_copy(k).start()
    for k in range(_NCH):
        in_copy(k).wait()
        out_copy(k).start()
        j = k + _DEPTH
        if j < _NCH:
            if j - _NBUF >= 0:
                out_copy(j - _NBUF).wait()
            in_copy(j).start()
    for m in range(_NCH - _NBUF, _NCH):
        out_copy(m).wait()


_x_concat = pl.pallas_call(
    _x_body,
    in_specs=[pl.BlockSpec(memory_space=pl.ANY),
              pl.BlockSpec(memory_space=pl.ANY)],
    out_specs=pl.BlockSpec(memory_space=pl.ANY),
    out_shape=jax.ShapeDtypeStruct((1, F, TOT, E), jnp.float32),
    scratch_shapes=[
        pltpu.VMEM((_NBUF, _HR, E), jnp.float32),
        pltpu.SemaphoreType.DMA((_NBUF,)),
        pltpu.SemaphoreType.DMA((_NBUF,)),
    ],
)


def kernel(embedding_X, embedding_y, tune_X, tune_y_table, labels):
    modifiedy = _y_concat()(tune_y_table, labels, embedding_y)
    modifiedX = jnp.transpose(
        _x_concat(jnp.transpose(tune_X, (0, 2, 1, 3)),
                  jnp.transpose(embedding_X, (0, 2, 1, 3))),
        (0, 2, 1, 3))
    return (modifiedX, modifiedy)
